# baseline (speedup 1.0000x reference)
import jax
import jax.numpy as jnp
from jax import lax
from jax.experimental import pallas as pl
from jax.experimental.pallas import tpu as pltpu

B, C, H, W = 32, 256, 32, 32
M, E, K = 4, 64, 2
I = E * H * W
HW = H * W
IBLK = 8192
IK = I // IBLK

ROWS = B * M * K
JJ = M * K


def _router_body(x_ref, w_ref, loss_ref, idx_ref, gate_ref, acc_ref):
    m = pl.program_id(0)
    k = pl.program_id(1)

    @pl.when(k == 0)
    def _init():
        acc_ref[pl.ds(m, 1)] = jnp.zeros((1, B, E), jnp.float32)

    xb = x_ref[:, 0, 0, :].astype(jnp.bfloat16)
    wb = w_ref[0].astype(jnp.bfloat16)
    acc_ref[pl.ds(m, 1)] += lax.dot_general(
        xb, wb, (((1,), (1,)), ((), ())),
        preferred_element_type=jnp.float32,
    )[None]

    @pl.when((m == M - 1) & (k == IK - 1))
    def _route():
        logits = acc_ref[...]
        z = logits - jnp.max(logits, axis=2, keepdims=True)
        ez = jnp.exp(z)
        p = ez / jnp.sum(ez, axis=2, keepdims=True)
        iota = lax.broadcasted_iota(jnp.int32, (M, B, E), 2)
        m1 = jnp.max(p, axis=2, keepdims=True)
        i1 = jnp.min(jnp.where(p == m1, iota, E), axis=2, keepdims=True)
        mask1 = iota == i1
        p2 = jnp.where(mask1, -1.0, p)
        m2 = jnp.max(p2, axis=2, keepdims=True)
        i2 = jnp.min(jnp.where(p2 == m2, iota, E), axis=2, keepdims=True)
        denom = m1 + m2 + 1e-6
        g1 = m1 / denom
        g2 = m2 / denom
        oh1 = mask1.astype(jnp.float32)
        oh2 = (iota == i2).astype(jnp.float32)
        imp = jnp.sum(g1 * oh1 + g2 * oh2, axis=1)
        loadv = jnp.sum(oh1 + oh2, axis=1)

        def cv2(v):
            n = M * E
            s = jnp.sum(v)
            ss = jnp.sum(v * v)
            mean = s / n
            var = (ss - n * mean * mean) / (n - 1)
            return var / (mean * mean + 1e-10)

        loss_ref[...] = jnp.reshape((cv2(imp) + cv2(loadv)) * 0.01, (1, 1))
        midx = lax.broadcasted_iota(jnp.int32, (M, B, 1), 0)
        base = midx * E
        idx_ref[:, :, 0:1] = base + i1
        idx_ref[:, :, 1:2] = base + i2
        gate_ref[:, :, 0:1] = g1
        gate_ref[:, :, 1:2] = g2


_router = pl.pallas_call(
    _router_body,
    grid=(M, IK),
    in_specs=[
        pl.BlockSpec((B, 1, 1, IBLK), lambda m, k: (0, m, 0, k)),
        pl.BlockSpec((1, E, IBLK), lambda m, k: (m, 0, k)),
    ],
    out_specs=[
        pl.BlockSpec((1, 1), lambda m, k: (0, 0)),
        pl.BlockSpec((M, B, K), lambda m, k: (0, 0, 0)),
        pl.BlockSpec((M, B, K), lambda m, k: (0, 0, 0)),
    ],
    out_shape=[
        jax.ShapeDtypeStruct((1, 1), jnp.float32),
        jax.ShapeDtypeStruct((M, B, K), jnp.int32),
        jax.ShapeDtypeStruct((M, B, K), jnp.float32),
    ],
    scratch_shapes=[pltpu.VMEM((M, B, E), jnp.float32)],
)


def _gather_body(idx_ref, *refs):
    xrows = refs[:JJ]
    arows = refs[JJ:2 * JJ]
    gates_ref = refs[2 * JJ]
    ox_ref = refs[2 * JJ + 1]
    oa_ref = refs[2 * JJ + 2]
    b = pl.program_id(0)
    sub_iota = lax.broadcasted_iota(jnp.int32, (1, 8, H, W), 1)
    for j in range(JJ):
        g = gates_ref[0, b * JJ + j]
        sub = idx_ref[b * JJ + j] % 8
        keep = sub_iota == sub
        ox_ref[0, j] = jnp.sum(jnp.where(keep, xrows[j][...], 0.0), axis=1)[0] * g
        oa_ref[0, j] = jnp.sum(jnp.where(keep, arows[j][...], 0.0), axis=1)[0] * g


def _make_in_spec(j):
    return pl.BlockSpec(
        (1, 8, H, W), lambda b, idx_ref, j=j: (b, idx_ref[b * JJ + j] // 8, 0, 0)
    )


_gather = pl.pallas_call(
    _gather_body,
    grid_spec=pltpu.PrefetchScalarGridSpec(
        num_scalar_prefetch=1,
        grid=(B,),
        in_specs=(
            [_make_in_spec(j) for j in range(JJ)]
            + [_make_in_spec(j) for j in range(JJ)]
            + [pl.BlockSpec(memory_space=pltpu.SMEM)]
        ),
        out_specs=[
            pl.BlockSpec((1, JJ, H, W), lambda b, idx_ref: (b, 0, 0, 0)),
            pl.BlockSpec((1, JJ, H, W), lambda b, idx_ref: (b, 0, 0, 0)),
        ],
    ),
    out_shape=[
        jax.ShapeDtypeStruct((B, JJ, H, W), jnp.float32),
        jax.ShapeDtypeStruct((B, JJ, H, W), jnp.float32),
    ],
)


def kernel(x, absolute_feature, w_gate):
    x4 = x.reshape(B, M, 1, I)
    wt = jnp.transpose(w_gate, (0, 2, 1))
    loss2, idx_mbk, gate_mbk = _router(x4, wt)
    loss = loss2[0, 0]
    idx_flat = jnp.transpose(idx_mbk, (1, 0, 2)).reshape(ROWS)
    gate_flat = jnp.transpose(gate_mbk, (1, 0, 2)).reshape(1, ROWS)
    args = [x] * JJ + [absolute_feature] * JJ + [gate_flat]
    wx, wa = _gather(idx_flat, *args)
    return (loss, wa, wx)


# speedup vs baseline: 3.3862x; 3.3862x over previous
import jax
import jax.numpy as jnp
from jax import lax
from jax.experimental import pallas as pl
from jax.experimental.pallas import tpu as pltpu

B, C, H, W = 32, 256, 32, 32
M, E, K = 4, 64, 2
I = E * H * W
HW = H * W
CBLK = 8
IBLK = CBLK * HW
IK = I // IBLK

ROWS = B * M * K
JJ = M * K


def _transpose_body(xt_ref, out_ref):
    out_ref[0] = jnp.transpose(xt_ref[0], (1, 0)).astype(jnp.bfloat16)


_transpose = pl.pallas_call(
    _transpose_body,
    grid=(B,),
    in_specs=[pl.BlockSpec((1, HW, C), lambda b: (b, 0, 0))],
    out_specs=pl.BlockSpec((1, C, HW), lambda b: (b, 0, 0)),
    out_shape=jax.ShapeDtypeStruct((B, C, HW), jnp.bfloat16),
)


def _router_body(x_ref, w_ref, loss_ref, idx_ref, gate_ref, acc_ref):
    m = pl.program_id(0)
    k = pl.program_id(1)

    part = jnp.zeros((B, E), jnp.float32)
    for cc in range(CBLK):
        xb = x_ref[:, cc, :]
        wb = w_ref[0, :, cc * HW:(cc + 1) * HW].astype(jnp.bfloat16)
        part += lax.dot_general(
            xb, wb, (((1,), (1,)), ((), ())),
            preferred_element_type=jnp.float32,
        )

    @pl.when(k == 0)
    def _init():
        acc_ref[pl.ds(m, 1)] = part[None]

    @pl.when(k != 0)
    def _acc():
        acc_ref[pl.ds(m, 1)] += part[None]

    @pl.when((m == M - 1) & (k == IK - 1))
    def _route():
        logits = acc_ref[...]
        z = logits - jnp.max(logits, axis=2, keepdims=True)
        ez = jnp.exp(z)
        p = ez / jnp.sum(ez, axis=2, keepdims=True)
        iota = lax.broadcasted_iota(jnp.int32, (M, B, E), 2)
        m1 = jnp.max(p, axis=2, keepdims=True)
        i1 = jnp.min(jnp.where(p == m1, iota, E), axis=2, keepdims=True)
        mask1 = iota == i1
        p2 = jnp.where(mask1, -1.0, p)
        m2 = jnp.max(p2, axis=2, keepdims=True)
        i2 = jnp.min(jnp.where(p2 == m2, iota, E), axis=2, keepdims=True)
        denom = m1 + m2 + 1e-6
        g1 = m1 / denom
        g2 = m2 / denom
        oh1 = mask1.astype(jnp.float32)
        oh2 = (iota == i2).astype(jnp.float32)
        imp = jnp.sum(g1 * oh1 + g2 * oh2, axis=1)
        loadv = jnp.sum(oh1 + oh2, axis=1)

        def cv2(v):
            n = M * E
            s = jnp.sum(v)
            ss = jnp.sum(v * v)
            mean = s / n
            var = (ss - n * mean * mean) / (n - 1)
            return var / (mean * mean + 1e-10)

        loss_ref[...] = jnp.reshape((cv2(imp) + cv2(loadv)) * 0.01, (1, 1))
        midx = lax.broadcasted_iota(jnp.int32, (M, B, 1), 0)
        base = midx * E
        idx_ref[:, :, 0:1] = base + i1
        idx_ref[:, :, 1:2] = base + i2
        gate_ref[:, :, 0:1] = g1
        gate_ref[:, :, 1:2] = g2


_router = pl.pallas_call(
    _router_body,
    grid=(M, IK),
    in_specs=[
        pl.BlockSpec((B, CBLK, HW), lambda m, k: (0, m * IK + k, 0)),
        pl.BlockSpec((1, E, IBLK), lambda m, k: (m, 0, k)),
    ],
    out_specs=[
        pl.BlockSpec((1, 1), lambda m, k: (0, 0)),
        pl.BlockSpec((M, B, K), lambda m, k: (0, 0, 0)),
        pl.BlockSpec((M, B, K), lambda m, k: (0, 0, 0)),
    ],
    out_shape=[
        jax.ShapeDtypeStruct((1, 1), jnp.float32),
        jax.ShapeDtypeStruct((M, B, K), jnp.int32),
        jax.ShapeDtypeStruct((M, B, K), jnp.float32),
    ],
    scratch_shapes=[pltpu.VMEM((M, B, E), jnp.float32)],
)


def _gather_body(idx_ref, x_ref, a_ref, ch_ref, g_ref, ox_ref, oa_ref):
    chv = ch_ref[0, 0, :]
    gv = g_ref[0, 0, :]
    cio = lax.broadcasted_iota(jnp.int32, (C, JJ), 0)
    oh = (cio == chv[None, :]).astype(jnp.float32) * gv[None, :]
    xb = x_ref[0]
    ab = a_ref[0]
    ox_ref[0] = lax.dot_general(
        oh, xb, (((0,), (1,)), ((), ())), preferred_element_type=jnp.float32
    )
    oa_ref[0] = lax.dot_general(
        oh, ab, (((0,), (1,)), ((), ())), preferred_element_type=jnp.float32
    )


_gather = pl.pallas_call(
    _gather_body,
    grid_spec=pltpu.PrefetchScalarGridSpec(
        num_scalar_prefetch=1,
        grid=(B,),
        in_specs=[
            pl.BlockSpec((1, HW, C), lambda b, idx_ref: (b, 0, 0)),
            pl.BlockSpec((1, HW, C), lambda b, idx_ref: (b, 0, 0)),
            pl.BlockSpec((1, 1, JJ), lambda b, idx_ref: (b, 0, 0)),
            pl.BlockSpec((1, 1, JJ), lambda b, idx_ref: (b, 0, 0)),
        ],
        out_specs=[
            pl.BlockSpec((1, JJ, HW), lambda b, idx_ref: (b, 0, 0)),
            pl.BlockSpec((1, JJ, HW), lambda b, idx_ref: (b, 0, 0)),
        ],
    ),
    out_shape=[
        jax.ShapeDtypeStruct((B, JJ, HW), jnp.float32),
        jax.ShapeDtypeStruct((B, JJ, HW), jnp.float32),
    ],
)


def kernel(x, absolute_feature, w_gate):
    xt = jnp.transpose(x, (0, 2, 3, 1)).reshape(B, HW, C)
    at = jnp.transpose(absolute_feature, (0, 2, 3, 1)).reshape(B, HW, C)
    xs = _transpose(xt)
    xs4 = xs.reshape(B, M * E, HW)
    wt = jnp.transpose(w_gate, (0, 2, 1))
    loss2, idx_mbk, gate_mbk = _router(xs4, wt)
    loss = loss2[0, 0]
    ch_b = jnp.transpose(idx_mbk, (1, 0, 2)).reshape(B, 1, JJ)
    gate_b = jnp.transpose(gate_mbk, (1, 0, 2)).reshape(B, 1, JJ)
    idx_flat = ch_b.reshape(ROWS)
    wx, wa = _gather(idx_flat, xt, at, ch_b, gate_b)
    return (loss, wa.reshape(B, JJ, H, W), wx.reshape(B, JJ, H, W))
